# baseline (speedup 1.0000x reference)
import functools
import numpy as np
import jax
import jax.numpy as jnp
from jax import lax
from jax.experimental import pallas as pl
from jax.experimental.pallas import tpu as pltpu
from jax.experimental.pallas import tpu_sc as plsc

_K = 64
_B = 128
_N = 32768
_NC = 2
_NS = 16
_NW = _NC * _NS
_RPW = _B // _NW
_NV = _N // 16
_NB = 256
_UNROLL = 8
_BANK = _NB * 16
_IMIN = np.int32(-2147483648)


def _sc_body(x_hbm, out_hbm, row_v, key_v, hist_v, suf_v):
    wid = lax.axis_index("s") * _NC + lax.axis_index("c")
    lanes = lax.iota(jnp.int32, 16)
    lane_base = lanes * _NB
    lane_bases = [lane_base + np.int32(j * _BANK) for j in range(_UNROLL)]
    ones16 = jnp.ones((16,), jnp.int32)

    def row_body(rr, _carry):
        row = wid * _RPW + rr
        pltpu.sync_copy(x_hbm.at[row], row_v)

        p_and_q = (jnp.int32(0), jnp.int32(_K))
        cnt_eq = jnp.int32(0)
        for p in range(4):
            pref, quota = p_and_q
            dshift = 24 - 8 * p
            eshift = 32 - 8 * p

            def zero_body(i, _):
                for j in range(8):
                    hist_v[pl.ds(i * 128 + j * 16, 16)] = jnp.zeros(
                        (16,), jnp.int32)
                return 0

            lax.fori_loop(0, (_NB * _UNROLL) // 8, zero_body, 0)

            def hist_body(i, _, _pref=pref, _dshift=dshift, _eshift=eshift):
                for j in range(_UNROLL):
                    o = i * (16 * _UNROLL) + j * 16
                    if p == 0:
                        xv = row_v[pl.ds(o, 16)]
                        bits = lax.bitcast_convert_type(xv, jnp.int32)
                        mneg = lax.shift_right_arithmetic(bits, 31)
                        u = bits ^ (mneg | _IMIN)
                        key_v[pl.ds(o, 16)] = u
                        elig = u == u
                    else:
                        u = key_v[pl.ds(o, 16)]
                        elig = lax.shift_right_logical(u, _eshift) == _pref
                    d = lax.shift_right_logical(u, _dshift) & jnp.int32(0xFF)
                    plsc.addupdate_scatter(hist_v, [lane_bases[j] + d],
                                           ones16, mask=elig)
                return 0

            lax.fori_loop(0, _NV // _UNROLL, hist_body, 0)

            def fold_body(i, _):
                for jj in range(4):
                    c16 = i * 64 + jj * 16
                    acc = hist_v[pl.ds(c16, 16)]
                    for b in range(1, _UNROLL):
                        acc = acc + hist_v[pl.ds(b * _BANK + c16, 16)]
                    hist_v[pl.ds(c16, 16)] = acc
                return 0

            lax.fori_loop(0, _BANK // 64, fold_body, 0)

            def suf_body(i, s):
                c = 15 - i
                tot = hist_v[pl.ds(c * 16, 16)]
                for l in range(1, 16):
                    tot = tot + hist_v[pl.ds(l * _NB + c * 16, 16)]
                cs = plsc.cumsum(tot)
                tchunk = jnp.sum(tot)
                suf_v[pl.ds(c * 16, 16)] = (tchunk - cs) + tot + s
                return s + tchunk

            lax.fori_loop(0, 16, suf_body, jnp.int32(0))

            def dsel_body(c, dmax, _quota=quota):
                sufc = suf_v[pl.ds(c * 16, 16)]
                cand = jnp.where(sufc >= _quota, c * 16 + lanes + 1,
                                 jnp.int32(0))
                return jnp.maximum(dmax, jnp.max(cand))

            d_star = lax.fori_loop(0, 16, dsel_body, jnp.int32(0)) - 1

            def ext_body(c, acc, _d=d_star):
                sufc = suf_v[pl.ds(c * 16, 16)]
                dd = c * 16 + lanes
                a = jnp.where(dd == _d + 1, sufc, jnp.int32(0))
                b = jnp.where(dd == _d, sufc, jnp.int32(0))
                return (jnp.maximum(acc[0], jnp.max(a)),
                        jnp.maximum(acc[1], jnp.max(b)))

            above, at_d = lax.fori_loop(
                0, 16, ext_body, (jnp.int32(0), jnp.int32(0)))
            if p == 3:
                cnt_eq = at_d - above
            p_and_q = ((pref << 8) | d_star, quota - above)

        t_pat, r = p_and_q
        t_s = t_pat ^ _IMIN

        def fast_body(i, _):
            for j in range(_UNROLL):
                o = i * (16 * _UNROLL) + j * 16
                v = key_v[pl.ds(o, 16)] ^ _IMIN
                xv = row_v[pl.ds(o, 16)]
                row_v[pl.ds(o, 16)] = jnp.where(
                    v >= t_s, xv, jnp.float32(0.0))
            return 0

        def tie_body(i, c):
            v = key_v[pl.ds(i * 16, 16)] ^ _IMIN
            xv = row_v[pl.ds(i * 16, 16)]
            eq = v == t_s
            pre = plsc.cumsum(jnp.where(eq, jnp.int32(1), jnp.int32(0)))
            keep = (v > t_s) | (eq & ((c + pre) <= r))
            row_v[pl.ds(i * 16, 16)] = jnp.where(keep, xv, jnp.float32(0.0))
            return c + jnp.max(pre)

        def fast_fn(_):
            lax.fori_loop(0, _NV // _UNROLL, fast_body, 0)
            return 0

        def tie_fn(_):
            lax.fori_loop(0, _NV, tie_body, jnp.int32(0))
            return 0

        lax.cond(cnt_eq > r, tie_fn, fast_fn, 0)
        pltpu.sync_copy(row_v, out_hbm.at[row])
        return 0

    lax.fori_loop(0, _RPW, row_body, 0)


@jax.jit
def kernel(x):
    mesh = plsc.VectorSubcoreMesh(core_axis_name="c", subcore_axis_name="s")
    return pl.kernel(
        _sc_body,
        out_type=jax.ShapeDtypeStruct((_B, _N), jnp.float32),
        mesh=mesh,
        compiler_params=pltpu.CompilerParams(needs_layout_passes=False),
        scratch_types=[
            pltpu.VMEM((_N,), jnp.float32),
            pltpu.VMEM((_N,), jnp.int32),
            pltpu.VMEM((_BANK * _UNROLL,), jnp.int32),
            pltpu.VMEM((_NB,), jnp.int32),
        ],
    )(x)


# speedup vs baseline: 2.4538x; 2.4538x over previous
import functools
import numpy as np
import jax
import jax.numpy as jnp
from jax import lax
from jax.experimental import pallas as pl
from jax.experimental.pallas import tpu as pltpu
from jax.experimental.pallas import tpu_sc as plsc

_K = 64
_B = 128
_N = 32768
_NC = 2
_NS = 16
_NW = _NC * _NS
_RPW = _B // _NW
_NV = _N // 16
_NB = 256
_UNROLL = 8
_BANK = _NB * 16
_IMIN = np.int32(-2147483648)


def _sc_body(x_hbm, out_hbm, row_v, key_v, hist_v, suf_v):
    wid = lax.axis_index("s") * _NC + lax.axis_index("c")
    lanes = lax.iota(jnp.int32, 16)
    lane_base = lanes * _NB
    lane_bases = [lane_base + np.int32(j * _BANK) for j in range(_UNROLL)]
    ones16 = jnp.ones((16,), jnp.int32)

    def row_body(rr, _carry):
        row = wid * _RPW + rr
        pltpu.sync_copy(x_hbm.at[row], row_v)

        p_and_q = (jnp.int32(0), jnp.int32(_K))
        cnt_eq = jnp.int32(0)
        for p in range(4):
            pref, quota = p_and_q
            dshift = 24 - 8 * p
            eshift = 32 - 8 * p

            @plsc.parallel_loop(0, (_NB * _UNROLL) // 16, unroll=8)
            def zero_body(i):
                hist_v[pl.ds(i * 16, 16)] = jnp.zeros((16,), jnp.int32)

            @plsc.parallel_loop(0, _NV, unroll=_UNROLL)
            def hist_body(i, _pref=pref, _dshift=dshift, _eshift=eshift):
                o = i * 16
                if p == 0:
                    xv = row_v[pl.ds(o, 16)]
                    bits = lax.bitcast_convert_type(xv, jnp.int32)
                    mneg = lax.shift_right_arithmetic(bits, 31)
                    u = bits ^ (mneg | _IMIN)
                    key_v[pl.ds(o, 16)] = u
                    elig = u == u
                else:
                    u = key_v[pl.ds(o, 16)]
                    elig = lax.shift_right_logical(u, _eshift) == _pref
                d = lax.shift_right_logical(u, _dshift) & jnp.int32(0xFF)
                bank = jnp.left_shift(i & jnp.int32(_UNROLL - 1),
                                      jnp.int32(12))
                plsc.addupdate_scatter(hist_v, [(lane_base + bank) + d],
                                       ones16, mask=elig)

            @plsc.parallel_loop(0, _BANK // 16, unroll=4)
            def fold_body(i):
                c16 = i * 16
                acc = hist_v[pl.ds(c16, 16)]
                for b in range(1, _UNROLL):
                    acc = acc + hist_v[pl.ds(b * _BANK + c16, 16)]
                hist_v[pl.ds(c16, 16)] = acc

            def suf_body(i, s):
                c = 15 - i
                tot = hist_v[pl.ds(c * 16, 16)]
                for l in range(1, 16):
                    tot = tot + hist_v[pl.ds(l * _NB + c * 16, 16)]
                cs = plsc.cumsum(tot)
                tchunk = jnp.sum(tot)
                suf_v[pl.ds(c * 16, 16)] = (tchunk - cs) + tot + s
                return s + tchunk

            lax.fori_loop(0, 16, suf_body, jnp.int32(0))

            def dsel_body(c, dmax, _quota=quota):
                sufc = suf_v[pl.ds(c * 16, 16)]
                cand = jnp.where(sufc >= _quota, c * 16 + lanes + 1,
                                 jnp.int32(0))
                return jnp.maximum(dmax, jnp.max(cand))

            d_star = lax.fori_loop(0, 16, dsel_body, jnp.int32(0)) - 1

            def ext_body(c, acc, _d=d_star):
                sufc = suf_v[pl.ds(c * 16, 16)]
                dd = c * 16 + lanes
                a = jnp.where(dd == _d + 1, sufc, jnp.int32(0))
                b = jnp.where(dd == _d, sufc, jnp.int32(0))
                return (jnp.maximum(acc[0], jnp.max(a)),
                        jnp.maximum(acc[1], jnp.max(b)))

            above, at_d = lax.fori_loop(
                0, 16, ext_body, (jnp.int32(0), jnp.int32(0)))
            if p == 3:
                cnt_eq = at_d - above
            p_and_q = ((pref << 8) | d_star, quota - above)

        t_pat, r = p_and_q
        t_s = t_pat ^ _IMIN

        def tie_body(i, c):
            v = key_v[pl.ds(i * 16, 16)] ^ _IMIN
            xv = row_v[pl.ds(i * 16, 16)]
            eq = v == t_s
            pre = plsc.cumsum(jnp.where(eq, jnp.int32(1), jnp.int32(0)))
            keep = (v > t_s) | (eq & ((c + pre) <= r))
            row_v[pl.ds(i * 16, 16)] = jnp.where(keep, xv, jnp.float32(0.0))
            return c + jnp.max(pre)

        def fast_fn(_):
            @plsc.parallel_loop(0, _NV, unroll=_UNROLL)
            def fast_body(i):
                o = i * 16
                v = key_v[pl.ds(o, 16)] ^ _IMIN
                xv = row_v[pl.ds(o, 16)]
                row_v[pl.ds(o, 16)] = jnp.where(
                    v >= t_s, xv, jnp.float32(0.0))

            return 0

        def tie_fn(_):
            lax.fori_loop(0, _NV, tie_body, jnp.int32(0))
            return 0

        lax.cond(cnt_eq > r, tie_fn, fast_fn, 0)
        pltpu.sync_copy(row_v, out_hbm.at[row])
        return 0

    lax.fori_loop(0, _RPW, row_body, 0)


@jax.jit
def kernel(x):
    mesh = plsc.VectorSubcoreMesh(core_axis_name="c", subcore_axis_name="s")
    return pl.kernel(
        _sc_body,
        out_type=jax.ShapeDtypeStruct((_B, _N), jnp.float32),
        mesh=mesh,
        compiler_params=pltpu.CompilerParams(needs_layout_passes=False),
        scratch_types=[
            pltpu.VMEM((_N,), jnp.float32),
            pltpu.VMEM((_N,), jnp.int32),
            pltpu.VMEM((_BANK * _UNROLL,), jnp.int32),
            pltpu.VMEM((_NB,), jnp.int32),
        ],
    )(x)


# speedup vs baseline: 2.5542x; 1.0409x over previous
import functools
import numpy as np
import jax
import jax.numpy as jnp
from jax import lax
from jax.experimental import pallas as pl
from jax.experimental.pallas import tpu as pltpu
from jax.experimental.pallas import tpu_sc as plsc

_K = 64
_B = 128
_N = 32768
_NC = 2
_NS = 16
_NW = _NC * _NS
_RPW = _B // _NW
_NV = _N // 16
_NB = 256
_UNROLL = 8
_BANK = _NB * 16
_IMIN = np.int32(-2147483648)


def _sc_body(x_hbm, out_hbm, row_v, key_v, hist_v, suf_v):
    wid = lax.axis_index("s") * _NC + lax.axis_index("c")
    lanes = lax.iota(jnp.int32, 16)
    lane_base = lanes * _NB
    lane_bases = [lane_base + np.int32(j * _BANK) for j in range(_UNROLL)]
    ones16 = jnp.ones((16,), jnp.int32)

    def row_body(rr, _carry):
        row = wid * _RPW + rr
        pltpu.sync_copy(x_hbm.at[row], row_v)

        p_and_q = (jnp.int32(0), jnp.int32(_K))
        cnt_eq = jnp.int32(0)
        for p in range(4):
            pref, quota = p_and_q
            dshift = 24 - 8 * p
            eshift = 32 - 8 * p

            @plsc.parallel_loop(0, (_BANK * _UNROLL) // 16, unroll=8)
            def zero_body(i):
                hist_v[pl.ds(i * 16, 16)] = jnp.zeros((16,), jnp.int32)

            @plsc.parallel_loop(0, _NV, unroll=_UNROLL)
            def hist_body(i, _pref=pref, _dshift=dshift, _eshift=eshift):
                o = i * 16
                if p == 0:
                    xv = row_v[pl.ds(o, 16)]
                    bits = lax.bitcast_convert_type(xv, jnp.int32)
                    mneg = lax.shift_right_arithmetic(bits, 31)
                    u = bits ^ (mneg | _IMIN)
                    key_v[pl.ds(o, 16)] = u
                    elig = u == u
                else:
                    u = key_v[pl.ds(o, 16)]
                    elig = lax.shift_right_logical(u, _eshift) == _pref
                d = lax.shift_right_logical(u, _dshift) & jnp.int32(0xFF)
                bank = jnp.left_shift(i & jnp.int32(_UNROLL - 1),
                                      jnp.int32(12))
                plsc.addupdate_scatter(hist_v, [(lane_base + bank) + d],
                                       ones16, mask=elig)

            @plsc.parallel_loop(0, _BANK // 16, unroll=4)
            def fold_body(i):
                c16 = i * 16
                acc = hist_v[pl.ds(c16, 16)]
                for b in range(1, _UNROLL):
                    acc = acc + hist_v[pl.ds(b * _BANK + c16, 16)]
                hist_v[pl.ds(c16, 16)] = acc

            def suf_body(i, s):
                c = 15 - i
                tot = hist_v[pl.ds(c * 16, 16)]
                for l in range(1, 16):
                    tot = tot + hist_v[pl.ds(l * _NB + c * 16, 16)]
                cs = plsc.cumsum(tot)
                tchunk = jnp.sum(tot)
                suf_v[pl.ds(c * 16, 16)] = (tchunk - cs) + tot + s
                return s + tchunk

            lax.fori_loop(0, 16, suf_body, jnp.int32(0))

            def dsel_body(c, dmax, _quota=quota):
                sufc = suf_v[pl.ds(c * 16, 16)]
                cand = jnp.where(sufc >= _quota, c * 16 + lanes + 1,
                                 jnp.int32(0))
                return jnp.maximum(dmax, jnp.max(cand))

            d_star = lax.fori_loop(0, 16, dsel_body, jnp.int32(0)) - 1

            def ext_body(c, acc, _d=d_star):
                sufc = suf_v[pl.ds(c * 16, 16)]
                dd = c * 16 + lanes
                a = jnp.where(dd == _d + 1, sufc, jnp.int32(0))
                b = jnp.where(dd == _d, sufc, jnp.int32(0))
                return (jnp.maximum(acc[0], jnp.max(a)),
                        jnp.maximum(acc[1], jnp.max(b)))

            above, at_d = lax.fori_loop(
                0, 16, ext_body, (jnp.int32(0), jnp.int32(0)))
            if p == 3:
                cnt_eq = at_d - above
            p_and_q = ((pref << 8) | d_star, quota - above)

        t_pat, r = p_and_q
        t_s = t_pat ^ _IMIN

        def tie_body(i, c):
            v = key_v[pl.ds(i * 16, 16)] ^ _IMIN
            xv = row_v[pl.ds(i * 16, 16)]
            eq = v == t_s
            pre = plsc.cumsum(jnp.where(eq, jnp.int32(1), jnp.int32(0)))
            keep = (v > t_s) | (eq & ((c + pre) <= r))
            row_v[pl.ds(i * 16, 16)] = jnp.where(keep, xv, jnp.float32(0.0))
            return c + jnp.max(pre)

        def fast_fn(_):
            @plsc.parallel_loop(0, _NV, unroll=_UNROLL)
            def fast_body(i):
                o = i * 16
                v = key_v[pl.ds(o, 16)] ^ _IMIN
                xv = row_v[pl.ds(o, 16)]
                row_v[pl.ds(o, 16)] = jnp.where(
                    v >= t_s, xv, jnp.float32(0.0))

            return 0

        def tie_fn(_):
            lax.fori_loop(0, _NV, tie_body, jnp.int32(0))
            return 0

        lax.cond(cnt_eq > r, tie_fn, fast_fn, 0)
        pltpu.sync_copy(row_v, out_hbm.at[row])
        return 0

    lax.fori_loop(0, _RPW, row_body, 0)


@jax.jit
def kernel(x):
    mesh = plsc.VectorSubcoreMesh(core_axis_name="c", subcore_axis_name="s")
    return pl.kernel(
        _sc_body,
        out_type=jax.ShapeDtypeStruct((_B, _N), jnp.float32),
        mesh=mesh,
        compiler_params=pltpu.CompilerParams(needs_layout_passes=False),
        scratch_types=[
            pltpu.VMEM((_N,), jnp.float32),
            pltpu.VMEM((_N,), jnp.int32),
            pltpu.VMEM((_BANK * _UNROLL,), jnp.int32),
            pltpu.VMEM((_NB,), jnp.int32),
        ],
    )(x)
